# baseline (speedup 1.0000x reference)
import jax
import jax.numpy as jnp
from jax import lax
from jax.experimental import pallas as pl
from jax.experimental.pallas import tpu as pltpu
from jax.experimental.pallas import tpu_sc as plsc

EMBED_DIM = 128


def _emb_body(idx_hbm, table_hbm, out_hbm, idx_v, rows_v, sem, wsem):
    info = plsc.get_sparse_core_info()
    nw = info.num_cores * info.num_subcores
    seq_len = idx_hbm.shape[1]
    per = (idx_hbm.shape[0] * seq_len) // nw
    tiles_per_row = seq_len // per
    wid = lax.axis_index("s") * info.num_cores + lax.axis_index("c")
    b = wid // tiles_per_row
    c0 = (wid % tiles_per_row) * per
    half = per // 2
    pltpu.sync_copy(idx_hbm.at[b, pl.ds(c0, per)], idx_v)
    ga = pltpu.async_copy(
        table_hbm.at[idx_v.at[pl.ds(0, half)]], rows_v.at[pl.ds(0, half)], sem
    )
    ga.wait()
    wa = pltpu.async_copy(
        rows_v.at[pl.ds(0, half)], out_hbm.at[b, pl.ds(c0, half)], wsem
    )
    gb = pltpu.async_copy(
        table_hbm.at[idx_v.at[pl.ds(half, half)]], rows_v.at[pl.ds(half, half)], sem
    )
    gb.wait()
    wb = pltpu.async_copy(
        rows_v.at[pl.ds(half, half)], out_hbm.at[b, pl.ds(c0 + half, half)], wsem
    )
    wa.wait()
    wb.wait()


def kernel(input_ids, embedding_weight):
    batch, seq_len = input_ids.shape
    mesh = plsc.VectorSubcoreMesh(core_axis_name="c", subcore_axis_name="s")
    info = plsc.get_sparse_core_info()
    per = (batch * seq_len) // (info.num_cores * info.num_subcores)
    return pl.kernel(
        _emb_body,
        mesh=mesh,
        out_type=jax.ShapeDtypeStruct((batch, seq_len, EMBED_DIM), jnp.float32),
        scratch_types=[
            pltpu.VMEM((per,), jnp.int32),
            pltpu.VMEM((per, EMBED_DIM), jnp.float32),
            pltpu.SemaphoreType.DMA,
            pltpu.SemaphoreType.DMA,
        ],
    )(input_ids.astype(jnp.int32), embedding_weight)


# speedup vs baseline: 1.0379x; 1.0379x over previous
import jax
import jax.numpy as jnp
from jax import lax
from jax.experimental import pallas as pl
from jax.experimental.pallas import tpu as pltpu
from jax.experimental.pallas import tpu_sc as plsc

EMBED_DIM = 128


def _emb_body(idx_hbm, table_hbm, out_hbm, idx_v, rows_v, sem):
    info = plsc.get_sparse_core_info()
    nw = info.num_cores * info.num_subcores
    seq_len = idx_hbm.shape[1]
    per = (idx_hbm.shape[0] * seq_len) // nw
    tiles_per_row = seq_len // per
    wid = lax.axis_index("s") * info.num_cores + lax.axis_index("c")
    b = wid // tiles_per_row
    c0 = (wid % tiles_per_row) * per
    pltpu.sync_copy(idx_hbm.at[b, pl.ds(c0, per)], idx_v)
    pltpu.async_copy(table_hbm.at[idx_v], rows_v, sem).wait()
    pltpu.sync_copy(rows_v, out_hbm.at[b, pl.ds(c0, per)])


def kernel(input_ids, embedding_weight):
    batch, seq_len = input_ids.shape
    mesh = plsc.VectorSubcoreMesh(core_axis_name="c", subcore_axis_name="s")
    info = plsc.get_sparse_core_info()
    per = (batch * seq_len) // (info.num_cores * info.num_subcores)
    return pl.kernel(
        _emb_body,
        mesh=mesh,
        out_type=jax.ShapeDtypeStruct((batch, seq_len, EMBED_DIM), jnp.float32),
        scratch_types=[
            pltpu.VMEM((per,), jnp.int32),
            pltpu.VMEM((per, EMBED_DIM), jnp.float32),
            pltpu.SemaphoreType.DMA,
        ],
    )(input_ids.astype(jnp.int32), embedding_weight)
